# baseline (speedup 1.0000x reference)
import jax
import jax.numpy as jnp
from jax.experimental import pallas as pl


_BLOCK_ROWS = 2000


def _fused_mlp_kernel(x_ref, w_in_ref, w_h1_ref, w_h2_ref, w_out_ref,
                      w_cls_ref, b_cls_ref, out_ref):
    x = x_ref[...]
    h = jax.nn.relu(jnp.dot(x, w_in_ref[...], preferred_element_type=jnp.float32))
    h = jax.nn.relu(jnp.dot(h, w_h1_ref[...], preferred_element_type=jnp.float32))
    h = jax.nn.relu(jnp.dot(h, w_h2_ref[...], preferred_element_type=jnp.float32))
    h = jnp.dot(h, w_out_ref[...], preferred_element_type=jnp.float32)
    y = jnp.dot(h, w_cls_ref[...], preferred_element_type=jnp.float32)
    out_ref[...] = y + b_cls_ref[...]


def kernel(h_0, W_in, W_h1, W_h2, W_out, W_cls, b_cls):
    B, N, D_in = h_0.shape
    D_h = W_in.shape[1]
    D_out = W_cls.shape[1]
    rows = B * N
    x = h_0.reshape(rows, D_in)
    b2 = b_cls.reshape(1, D_out)

    block_rows = _BLOCK_ROWS if rows % _BLOCK_ROWS == 0 else rows
    grid = (rows // block_rows,)

    def w_spec(shape):
        return pl.BlockSpec(shape, lambda i: (0, 0))

    y = pl.pallas_call(
        _fused_mlp_kernel,
        grid=grid,
        in_specs=[
            pl.BlockSpec((block_rows, D_in), lambda i: (i, 0)),
            w_spec((D_in, D_h)),
            w_spec((D_h, D_h)),
            w_spec((D_h, D_h)),
            w_spec((D_h, D_h)),
            w_spec((D_h, D_out)),
            w_spec((1, D_out)),
        ],
        out_specs=pl.BlockSpec((block_rows, D_out), lambda i: (i, 0)),
        out_shape=jax.ShapeDtypeStruct((rows, D_out), jnp.float32),
    )(x, W_in, W_h1, W_h2, W_out, W_cls, b2)

    return y.reshape(B, N, D_out)
